# baseline (speedup 1.0000x reference)
import functools

import jax
import jax.numpy as jnp
from jax import lax
from jax.experimental import pallas as pl
from jax.experimental.pallas import tpu as pltpu
from jax.experimental.pallas import tpu_sc as plsc

C = 128
C2 = C // 2
N = 10000
K = 32
NC, NS = 2, 16
NW = NC * NS
T_P = C2 // NW
CH_N = 400
CH_NG = CH_N // 16
N_CHUNK = N // CH_N

MASK_HI = -65536

TC_BLK = N
TC_GRID = 1


def _h_body(x_ref, w1_ref, b1_ref, hp_ref):
    h = lax.dot_general(w1_ref[...], x_ref[...],
                        dimension_numbers=(((1,), (0,)), ((), ())),
                        preferred_element_type=jnp.float32)
    h = jnp.maximum(h + b1_ref[...], 0.0)
    lo = lax.bitcast_convert_type(
        h[:C2, :].astype(jnp.bfloat16).astype(jnp.float32), jnp.int32)
    hi = lax.bitcast_convert_type(
        h[C2:, :].astype(jnp.bfloat16).astype(jnp.float32), jnp.int32)
    hp_ref[...] = (lo >> 16) | (hi & MASK_HI)


def _compute_hp(x_cn, w1, b1):
    return pl.pallas_call(
        _h_body,
        grid=(TC_GRID,),
        in_specs=[
            pl.BlockSpec((C, TC_BLK), lambda i: (0, i)),
            pl.BlockSpec((C, C), lambda i: (0, 0)),
            pl.BlockSpec((C, 1), lambda i: (0, 0)),
        ],
        out_specs=pl.BlockSpec((C2, TC_BLK), lambda i: (0, i)),
        out_shape=jax.ShapeDtypeStruct((C2, N), jnp.int32),
    )(x_cn, w1, b1.reshape(C, 1))


@functools.cache
def _make_sc_gather_max():
    @functools.partial(
        pl.kernel,
        out_type=jax.ShapeDtypeStruct((C2, N_CHUNK, CH_NG, 16), jnp.int32),
        mesh=plsc.VectorSubcoreMesh(core_axis_name="c", subcore_axis_name="s"),
        compiler_params=pltpu.CompilerParams(needs_layout_passes=False,
                                             use_tc_tiling_on_sc=False),
        scratch_types=[
            pltpu.VMEM((T_P * N,), jnp.int32),
            pltpu.VMEM((2, CH_NG, K, 16), jnp.int32),
            pltpu.VMEM((2, T_P, CH_NG, 16), jnp.int32),
            pltpu.SemaphoreType.DMA,
            pltpu.SemaphoreType.DMA,
            pltpu.SemaphoreType.DMA,
            pltpu.SemaphoreType.DMA,
        ],
    )
    def _sc_gather_max(hp, idx4, out, table_v, idx_ring, out_ring,
                       sem_i0, sem_i1, sem_o0, sem_o1):
        t = lax.axis_index("s") * NC + lax.axis_index("c")
        sems_i = [sem_i0, sem_i1]
        sems_o = [sem_o0, sem_o1]

        for p in range(T_P):
            pltpu.sync_copy(hp.at[T_P * t + p],
                            table_v.at[pl.ds(p * N, N)])

        def start_idx(c, rb):
            pltpu.async_copy(idx4.at[c], idx_ring.at[rb], sems_i[rb])

        def drain_idx(rb):
            pltpu.make_async_copy(idx4.at[0], idx_ring.at[rb],
                                  sems_i[rb]).wait()

        def start_out(c, rb):
            for p in range(T_P):
                pltpu.async_copy(out_ring.at[rb, p], out.at[T_P * t + p, c],
                                 sems_o[rb])

        def drain_out(rb):
            for p in range(T_P):
                pltpu.make_async_copy(out_ring.at[rb, p], out.at[0, 0],
                                      sems_o[rb]).wait()

        start_idx(0, 0)

        def chunk_body(c, rb):
            @pl.when(c + 1 < N_CHUNK)
            def _():
                start_idx(c + 1, 1 - rb)

            drain_idx(rb)

            @pl.when(c >= 2)
            def _():
                drain_out(rb)

            def ng_body(ng, _):
                ivs = [idx_ring[rb, ng, k, :] for k in range(K)]
                for p in range(T_P):
                    fvs = [iv + (p * N) for iv in ivs] if p else ivs
                    v = plsc.load_gather(table_v, [fvs[0]])
                    acc_hi = v
                    acc_lo = v & 0xFFFF
                    for k in range(1, K):
                        v = plsc.load_gather(table_v, [fvs[k]])
                        acc_hi = jnp.maximum(acc_hi, v)
                        acc_lo = jnp.maximum(acc_lo, v & 0xFFFF)
                    out_ring[rb, p, ng, :] = (acc_hi & MASK_HI) | acc_lo
                return 0

            lax.fori_loop(0, CH_NG, ng_body, 0)
            start_out(c, rb)

        def chunk_pair(c2, _):
            for rb in range(2):
                c = 2 * c2 + rb

                @pl.when(c < N_CHUNK)
                def _():
                    chunk_body(c, rb)
            return 0

        lax.fori_loop(0, (N_CHUNK + 1) // 2, chunk_pair, 0)
        drain_out(0)
        drain_out(1)

    return _sc_gather_max


def _out_body(x_ref, mp_ref, w2a_ref, w2bl_ref, w2bh_ref, b2_ref, o_ref):
    mp = mp_ref[...]
    m_lo = lax.bitcast_convert_type(mp << 16, jnp.float32)
    m_hi = lax.bitcast_convert_type(mp & MASK_HI, jnp.float32)
    a = lax.dot_general(w2a_ref[...], x_ref[...],
                        dimension_numbers=(((1,), (0,)), ((), ())),
                        preferred_element_type=jnp.float32)
    b = lax.dot_general(w2bl_ref[...], m_lo,
                        dimension_numbers=(((1,), (0,)), ((), ())),
                        preferred_element_type=jnp.float32)
    d = lax.dot_general(w2bh_ref[...], m_hi,
                        dimension_numbers=(((1,), (0,)), ((), ())),
                        preferred_element_type=jnp.float32)
    o_ref[...] = jnp.maximum(a + b + d + b2_ref[...], 0.0)


def _compute_out(x_cn, mp_cn, w2a, w2bl, w2bh, b2):
    return pl.pallas_call(
        _out_body,
        grid=(TC_GRID,),
        in_specs=[
            pl.BlockSpec((C, TC_BLK), lambda i: (0, i)),
            pl.BlockSpec((C2, TC_BLK), lambda i: (0, i)),
            pl.BlockSpec((C, C), lambda i: (0, 0)),
            pl.BlockSpec((C, C2), lambda i: (0, 0)),
            pl.BlockSpec((C, C2), lambda i: (0, 0)),
            pl.BlockSpec((C, 1), lambda i: (0, 0)),
        ],
        out_specs=pl.BlockSpec((C, TC_BLK), lambda i: (0, i)),
        out_shape=jax.ShapeDtypeStruct((C, N), jnp.float32),
    )(x_cn, mp_cn, w2a, w2bl, w2bh, b2.reshape(C, 1))


def kernel(x, edge_index, W1, b1, W2, b2):
    x_cn = x[0, :, :, 0]
    idx = edge_index[0, 0].astype(jnp.int32)
    idx4 = idx.reshape(N_CHUNK, CH_NG, 16, K).transpose(0, 1, 3, 2)

    hp = _compute_hp(x_cn, W1, b1)
    mp_raw = _make_sc_gather_max()(hp, idx4)
    mp_cn = mp_raw.reshape(C2, N)
    w2b = W2[:, C:]
    out_cn = _compute_out(x_cn, mp_cn, W2[:, :C],
                          w2b[:, :C2], w2b[:, C2:], b2)
    return out_cn.reshape(1, C, N, 1)


# speedup vs baseline: 1.0173x; 1.0173x over previous
import functools

import jax
import jax.numpy as jnp
from jax import lax
from jax.experimental import pallas as pl
from jax.experimental.pallas import tpu as pltpu
from jax.experimental.pallas import tpu_sc as plsc

C = 128
N = 10000
K = 32
NC, NS = 2, 16
NW = NC * NS
T_F = C // NW
CH_N = 400
CH_NG = CH_N // 16
N_CHUNK = N // CH_N

TC_BLK = N
TC_GRID = 1


def _h_body(x_ref, w1_ref, b1_ref, h_ref):
    h = lax.dot_general(w1_ref[...], x_ref[...],
                        dimension_numbers=(((1,), (0,)), ((), ())),
                        preferred_element_type=jnp.float32)
    h_ref[...] = jnp.maximum(h + b1_ref[...], 0.0)


def _compute_h(x_cn, w1, b1):
    return pl.pallas_call(
        _h_body,
        grid=(TC_GRID,),
        in_specs=[
            pl.BlockSpec((C, TC_BLK), lambda i: (0, i)),
            pl.BlockSpec((C, C), lambda i: (0, 0)),
            pl.BlockSpec((C, 1), lambda i: (0, 0)),
        ],
        out_specs=pl.BlockSpec((C, TC_BLK), lambda i: (0, i)),
        out_shape=jax.ShapeDtypeStruct((C, N), jnp.float32),
    )(x_cn, w1, b1.reshape(C, 1))


@functools.cache
def _make_sc_gather_max():
    @functools.partial(
        pl.kernel,
        out_type=jax.ShapeDtypeStruct((NW, T_F, N_CHUNK, CH_NG, 16),
                                      jnp.float32),
        mesh=plsc.VectorSubcoreMesh(core_axis_name="c", subcore_axis_name="s"),
        compiler_params=pltpu.CompilerParams(needs_layout_passes=False,
                                             use_tc_tiling_on_sc=False),
        scratch_types=[
            pltpu.VMEM((N,), jnp.float32),
            pltpu.VMEM((N,), jnp.float32),
            pltpu.VMEM((N,), jnp.float32),
            pltpu.VMEM((N,), jnp.float32),
            pltpu.VMEM((2, CH_NG, K, 16), jnp.int32),
            pltpu.VMEM((2, T_F, CH_NG, 16), jnp.float32),
            pltpu.SemaphoreType.DMA,
            pltpu.SemaphoreType.DMA,
            pltpu.SemaphoreType.DMA,
            pltpu.SemaphoreType.DMA,
        ],
    )
    def _sc_gather_max(h, idx4, out, tab0, tab1, tab2, tab3,
                       idx_ring, out_ring,
                       sem_i0, sem_i1, sem_o0, sem_o1):
        t = lax.axis_index("s") * NC + lax.axis_index("c")
        sems_i = [sem_i0, sem_i1]
        sems_o = [sem_o0, sem_o1]
        tabs = [tab0, tab1, tab2, tab3]

        for f in range(T_F):
            pltpu.sync_copy(h.at[t * T_F + f], tabs[f])

        def start_idx(c, rb):
            pltpu.async_copy(idx4.at[c], idx_ring.at[rb], sems_i[rb])

        def drain_idx(rb):
            pltpu.make_async_copy(idx4.at[0], idx_ring.at[rb],
                                  sems_i[rb]).wait()

        def start_out(c, rb):
            for f in range(T_F):
                pltpu.async_copy(out_ring.at[rb, f], out.at[t, f, c],
                                 sems_o[rb])

        def drain_out(rb):
            for f in range(T_F):
                pltpu.make_async_copy(out_ring.at[rb, f], out.at[t, f, 0],
                                      sems_o[rb]).wait()

        start_idx(0, 0)

        def chunk_body(c, rb):
            @pl.when(c + 1 < N_CHUNK)
            def _():
                start_idx(c + 1, 1 - rb)

            drain_idx(rb)

            @pl.when(c >= 2)
            def _():
                drain_out(rb)

            def ng_body(ng, _):
                ivs = [idx_ring[rb, ng, k, :] for k in range(K)]
                for f in range(T_F):
                    acc = plsc.load_gather(tabs[f], [ivs[0]])
                    for k in range(1, K):
                        acc = jnp.maximum(
                            acc, plsc.load_gather(tabs[f], [ivs[k]]))
                    out_ring[rb, f, ng, :] = acc
                return 0

            lax.fori_loop(0, CH_NG, ng_body, 0)
            start_out(c, rb)

        def chunk_pair(c2, _):
            for rb in range(2):
                c = 2 * c2 + rb

                @pl.when(c < N_CHUNK)
                def _():
                    chunk_body(c, rb)
            return 0

        lax.fori_loop(0, (N_CHUNK + 1) // 2, chunk_pair, 0)
        drain_out(0)
        drain_out(1)

    return _sc_gather_max


def _out_body(x_ref, m_ref, w2a_ref, w2b_ref, b2_ref, o_ref):
    a = lax.dot_general(w2a_ref[...], x_ref[...],
                        dimension_numbers=(((1,), (0,)), ((), ())),
                        preferred_element_type=jnp.float32)
    b = lax.dot_general(w2b_ref[...], m_ref[...],
                        dimension_numbers=(((1,), (0,)), ((), ())),
                        preferred_element_type=jnp.float32)
    o_ref[...] = jnp.maximum(a + b + b2_ref[...], 0.0)


def _compute_out(x_cn, m_cn, w2a, w2b, b2):
    return pl.pallas_call(
        _out_body,
        grid=(TC_GRID,),
        in_specs=[
            pl.BlockSpec((C, TC_BLK), lambda i: (0, i)),
            pl.BlockSpec((C, TC_BLK), lambda i: (0, i)),
            pl.BlockSpec((C, C), lambda i: (0, 0)),
            pl.BlockSpec((C, C), lambda i: (0, 0)),
            pl.BlockSpec((C, 1), lambda i: (0, 0)),
        ],
        out_specs=pl.BlockSpec((C, TC_BLK), lambda i: (0, i)),
        out_shape=jax.ShapeDtypeStruct((C, N), jnp.float32),
    )(x_cn, m_cn, w2a, w2b, b2.reshape(C, 1))


def kernel(x, edge_index, W1, b1, W2, b2):
    x_cn = x[0, :, :, 0]
    idx = edge_index[0, 0].astype(jnp.int32)
    idx4 = idx.reshape(N_CHUNK, CH_NG, 16, K).transpose(0, 1, 3, 2)

    h = _compute_h(x_cn, W1, b1)
    m_raw = _make_sc_gather_max()(h, idx4)
    m_cn = m_raw.reshape(C, N)
    out_cn = _compute_out(x_cn, m_cn, W2[:, :C], W2[:, C:], b2)
    return out_cn.reshape(1, C, N, 1)
